# baseline (speedup 1.0000x reference)
import functools

import jax
import jax.numpy as jnp
from jax import lax
from jax.experimental import pallas as pl
from jax.experimental.pallas import tpu as pltpu
from jax.experimental.pallas import tpu_sc as plsc

N = 10000
E = 320000
C = 128
CQ = 32
EPS = 1e-5

BE = 4000
GE = E // BE

NC = 2
NS = 16
NW = NC * NS
EW = E // NW
CH = 80
NCH = EW // CH
NT = N // NS
NT0 = 624
NTT = N - NS * NT0

F32 = jnp.float32
BF16 = jnp.bfloat16


def _elu(x):
    return jnp.where(x > 0, x, jnp.exp(x) - 1.0)


def _k1_body(ef_ref, w_ref, b_ref, z_ref, st_ref):
    i = pl.program_id(0)
    z = jnp.dot(ef_ref[...], w_ref[...], preferred_element_type=F32) + b_ref[...]
    z_ref[...] = z.astype(BF16)

    @pl.when(i == 0)
    def _():
        st_ref[...] = jnp.zeros_like(st_ref)

    s = jnp.sum(z, axis=0, keepdims=True)
    sq = jnp.sum(z * z, axis=0, keepdims=True)
    st_ref[...] += jnp.concatenate([s, sq], axis=0)


def _k1(ef, w, b):
    return pl.pallas_call(
        _k1_body,
        grid=(GE,),
        in_specs=[
            pl.BlockSpec((BE, C), lambda i: (i, 0)),
            pl.BlockSpec((C, CQ), lambda i: (0, 0)),
            pl.BlockSpec((1, CQ), lambda i: (0, 0)),
        ],
        out_specs=[
            pl.BlockSpec((BE, CQ), lambda i: (i, 0)),
            pl.BlockSpec((2, CQ), lambda i: (0, 0)),
        ],
        out_shape=[
            jax.ShapeDtypeStruct((E, CQ), BF16),
            jax.ShapeDtypeStruct((2, CQ), F32),
        ],
    )(ef, w, b)


def _k2_body(nf_ref, wdn_ref, bdn_ref, g_ref, bb_ref,
             w1a_ref, w1b_ref, w2a_ref, be1_ref, be2_ref,
             hn_ref, ac_ref, b2_ref):
    nf = nf_ref[...]
    z = jnp.dot(nf, wdn_ref[...], preferred_element_type=F32) + bdn_ref[...]
    m = jnp.mean(z, axis=0, keepdims=True)
    v = jnp.mean(z * z, axis=0, keepdims=True) - m * m
    sc = g_ref[...] * lax.rsqrt(v + EPS)
    t = bb_ref[...] - m * sc
    hn = _elu(z * sc + t)
    hn_ref[...] = hn
    a = jnp.dot(hn, w1a_ref[...], preferred_element_type=F32) + be1_ref[...]
    cc = jnp.dot(hn, w2a_ref[...], preferred_element_type=F32) + be2_ref[...]
    ai = lax.bitcast_convert_type(a, jnp.int32) + 0x8000
    ci = lax.bitcast_convert_type(cc, jnp.int32) + 0x8000
    hi = jnp.bitwise_and(ai, jnp.int32(-65536))
    lo = jnp.bitwise_and(jnp.right_shift(ci, 16), jnp.int32(0xFFFF))
    ac_ref[...] = jnp.bitwise_or(hi, lo)
    b2_ref[...] = jnp.dot(hn, w1b_ref[...], preferred_element_type=F32)


def _k2(nf, wdn, bdn, g, bb, w1a, w1b, w2a, be1, be2):
    return pl.pallas_call(
        _k2_body,
        out_shape=[
            jax.ShapeDtypeStruct((N, CQ), F32),
            jax.ShapeDtypeStruct((N, CQ), jnp.int32),
            jax.ShapeDtypeStruct((N, CQ), F32),
        ],
    )(nf, wdn, bdn, g, bb, w1a, w1b, w2a, be1, be2)


NSL = 3


def _k3_body(ac_hbm, b_hbm, src_hbm, dst_hbm,
             gac_out, gb_out,
             tabac_sh, tabb_sh,
             idxs, idxd, bufac, bufb, isem, gsem, osem):
    c = lax.axis_index("c")
    s = lax.axis_index("s")
    wid = s * NC + c
    base = wid * EW

    @pl.when(s == 0)
    def _():
        pltpu.sync_copy(ac_hbm, tabac_sh)

    @pl.when(s == 1)
    def _():
        pltpu.sync_copy(b_hbm, tabb_sh)

    plsc.subcore_barrier()

    def islc(j):
        return pl.ds((j % NSL) * CH, CH)

    def ifire(j):
        sl = j % NSL
        pltpu.async_copy(src_hbm.at[pl.ds(base + j * CH, CH)],
                         idxs.at[islc(j)], isem.at[sl])
        pltpu.async_copy(dst_hbm.at[pl.ds(base + j * CH, CH)],
                         idxd.at[islc(j)], isem.at[sl])

    def iwait(j):
        sl = j % NSL
        pltpu.make_async_copy(src_hbm.at[pl.ds(base + j * CH, CH)],
                              idxs.at[islc(j)], isem.at[sl]).wait()
        pltpu.make_async_copy(dst_hbm.at[pl.ds(base + j * CH, CH)],
                              idxd.at[islc(j)], isem.at[sl]).wait()

    def gfire(j):
        sl = j % NSL
        pltpu.async_copy(tabac_sh.at[idxs.at[islc(j)]], bufac.at[islc(j)],
                         gsem.at[sl])
        pltpu.async_copy(tabb_sh.at[idxd.at[islc(j)]], bufb.at[islc(j)],
                         gsem.at[sl])

    def gwait(j):
        sl = j % NSL
        pltpu.make_async_copy(tabac_sh.at[idxs.at[islc(j)]],
                              bufac.at[islc(j)], gsem.at[sl]).wait()
        pltpu.make_async_copy(tabb_sh.at[idxd.at[islc(j)]],
                              bufb.at[islc(j)], gsem.at[sl]).wait()

    def ofire(j):
        sl = j % NSL
        off = pl.ds(base + j * CH, CH)
        pltpu.async_copy(bufac.at[islc(j)], gac_out.at[off], osem.at[sl])
        pltpu.async_copy(bufb.at[islc(j)], gb_out.at[off], osem.at[sl])

    def owait(j):
        sl = j % NSL
        off = pl.ds(base + j * CH, CH)
        pltpu.make_async_copy(bufac.at[islc(j)], gac_out.at[off],
                              osem.at[sl]).wait()
        pltpu.make_async_copy(bufb.at[islc(j)], gb_out.at[off],
                              osem.at[sl]).wait()

    ifire(0)
    ifire(1)
    iwait(0)
    gfire(0)

    def step(i, carry):
        j2 = i + 2
        j1 = i + 1

        @pl.when(j2 < NCH)
        def _():
            ifire(j2)

        @pl.when(j1 < NCH)
        def _():
            iwait(j1)

            @pl.when(j1 >= NSL)
            def _():
                owait(j1 - NSL)

            gfire(j1)

        gwait(i)
        ofire(i)
        return carry

    lax.fori_loop(0, NCH, step, 0)
    for t in range(NSL):
        owait(NCH - NSL + t)


def _k3(ac, b, src, dst):
    mesh = plsc.VectorSubcoreMesh(core_axis_name="c", subcore_axis_name="s")
    k = pl.kernel(
        _k3_body,
        out_type=[
            jax.ShapeDtypeStruct((E, CQ), jnp.int32),
            jax.ShapeDtypeStruct((E, CQ), F32),
        ],
        mesh=mesh,
        scratch_types=[
            pltpu.VMEM_SHARED((N, CQ), jnp.int32),
            pltpu.VMEM_SHARED((N, CQ), F32),
            pltpu.VMEM((NSL * CH,), jnp.int32),
            pltpu.VMEM((NSL * CH,), jnp.int32),
            pltpu.VMEM((NSL * CH, CQ), jnp.int32),
            pltpu.VMEM((NSL * CH, CQ), F32),
            pltpu.SemaphoreType.DMA((NSL,)),
            pltpu.SemaphoreType.DMA((NSL,)),
            pltpu.SemaphoreType.DMA((NSL,)),
        ],
    )
    return k(ac, b, src, dst)


def _k4_body(z_ref, gac_ref, gb_ref, ste_ref, g_ref, bb_ref, w1c_ref,
             y1_ref, st_ref):
    i = pl.program_id(0)
    m = ste_ref[0:1, :] * (1.0 / E)
    v = ste_ref[1:2, :] * (1.0 / E) - m * m
    sc = g_ref[...] * lax.rsqrt(v + EPS)
    t = bb_ref[...] - m * sc
    he = _elu(z_ref[...].astype(F32) * sc + t)
    ga = lax.bitcast_convert_type(
        jnp.bitwise_and(gac_ref[...], jnp.int32(-65536)), F32)
    y1 = jnp.dot(he, w1c_ref[...], preferred_element_type=F32) \
        + ga + gb_ref[...]
    y1_ref[...] = y1.astype(BF16)

    @pl.when(i == 0)
    def _():
        st_ref[...] = jnp.zeros_like(st_ref)

    s = jnp.sum(y1, axis=0, keepdims=True)
    sq = jnp.sum(y1 * y1, axis=0, keepdims=True)
    st_ref[...] += jnp.concatenate([s, sq], axis=0)


def _k4(z, gac, gb, ste, g, bb, w1c):
    return pl.pallas_call(
        _k4_body,
        grid=(GE,),
        in_specs=[
            pl.BlockSpec((BE, CQ), lambda i: (i, 0)),
            pl.BlockSpec((BE, CQ), lambda i: (i, 0)),
            pl.BlockSpec((BE, CQ), lambda i: (i, 0)),
            pl.BlockSpec((2, CQ), lambda i: (0, 0)),
            pl.BlockSpec((1, CQ), lambda i: (0, 0)),
            pl.BlockSpec((1, CQ), lambda i: (0, 0)),
            pl.BlockSpec((CQ, CQ), lambda i: (0, 0)),
        ],
        out_specs=[
            pl.BlockSpec((BE, CQ), lambda i: (i, 0)),
            pl.BlockSpec((2, CQ), lambda i: (0, 0)),
        ],
        out_shape=[
            jax.ShapeDtypeStruct((E, CQ), BF16),
            jax.ShapeDtypeStruct((2, CQ), F32),
        ],
    )(z, gac, gb, ste, g, bb, w1c)


def _k5_body(y1_ref, gac_ref, st1_ref, g1_ref, bb1_ref, w2b_ref,
             g2_ref, bb2_ref,
             ex_ref, y2_ref, gram_ref, sx_ref, st2_ref, acc_ref):
    i = pl.program_id(0)
    m = st1_ref[0:1, :] * (1.0 / E)
    v = st1_ref[1:2, :] * (1.0 / E) - m * m
    sc = g1_ref[...] * lax.rsqrt(v + EPS)
    t = bb1_ref[...] - m * sc
    ex = _elu(y1_ref[...].astype(F32) * sc + t)
    ex_ref[...] = ex.astype(BF16)
    gc = lax.bitcast_convert_type(
        jnp.left_shift(gac_ref[...], 16), F32)
    y2 = jnp.dot(ex, w2b_ref[...], preferred_element_type=F32) + gc
    y2_ref[...] = y2

    @pl.when(i == 0)
    def _():
        gram_ref[...] = jnp.zeros_like(gram_ref)
        sx_ref[...] = jnp.zeros_like(sx_ref)
        acc_ref[...] = jnp.zeros_like(acc_ref)

    gram_ref[...] += lax.dot_general(
        ex, ex, (((0,), (0,)), ((), ())), preferred_element_type=F32)
    sx_ref[...] += jnp.sum(ex, axis=0, keepdims=True)
    s = jnp.sum(y2, axis=0, keepdims=True)
    sq = jnp.sum(y2 * y2, axis=0, keepdims=True)
    acc_ref[...] += jnp.concatenate([s, sq], axis=0)

    @pl.when(i == GE - 1)
    def _():
        m2 = acc_ref[0:1, :] * (1.0 / E)
        v2 = acc_ref[1:2, :] * (1.0 / E) - m2 * m2
        sc2 = g2_ref[...] * lax.rsqrt(v2 + EPS)
        t2 = bb2_ref[...] - m2 * sc2
        st2_ref[...] = jnp.concatenate([sc2, t2], axis=0)


def _k5(y1, gac, st1, g1, bb1, w2b, g2, bb2):
    return pl.pallas_call(
        _k5_body,
        grid=(GE,),
        in_specs=[
            pl.BlockSpec((BE, CQ), lambda i: (i, 0)),
            pl.BlockSpec((BE, CQ), lambda i: (i, 0)),
            pl.BlockSpec((2, CQ), lambda i: (0, 0)),
            pl.BlockSpec((1, CQ), lambda i: (0, 0)),
            pl.BlockSpec((1, CQ), lambda i: (0, 0)),
            pl.BlockSpec((CQ, CQ), lambda i: (0, 0)),
            pl.BlockSpec((1, CQ), lambda i: (0, 0)),
            pl.BlockSpec((1, CQ), lambda i: (0, 0)),
        ],
        out_specs=[
            pl.BlockSpec((BE, CQ), lambda i: (i, 0)),
            pl.BlockSpec((BE, CQ), lambda i: (i, 0)),
            pl.BlockSpec((CQ, CQ), lambda i: (0, 0)),
            pl.BlockSpec((1, CQ), lambda i: (0, 0)),
            pl.BlockSpec((2, CQ), lambda i: (0, 0)),
        ],
        out_shape=[
            jax.ShapeDtypeStruct((E, CQ), BF16),
            jax.ShapeDtypeStruct((E, CQ), F32),
            jax.ShapeDtypeStruct((CQ, CQ), F32),
            jax.ShapeDtypeStruct((1, CQ), F32),
            jax.ShapeDtypeStruct((2, CQ), F32),
        ],
        scratch_shapes=[pltpu.VMEM((2, CQ), F32)],
    )(y1, gac, st1, g1, bb1, w2b, g2, bb2)


NBY = 4
KP6 = 2


def _k6_body(y2_hbm, dst3_hbm, st_hbm, ones_hbm, zacc_hbm, zcnt_hbm,
             seg_out, cnt_out,
             acc_sh, cnt_sh, dstbuf, ybuf, ones_v, st_v, ysem, msem):
    c = lax.axis_index("c")
    s = lax.axis_index("s")
    wid = s * NC + c
    tid = s
    lo = tid * NT0
    pltpu.sync_copy(zacc_hbm.at[pl.ds(0, NT0)], acc_sh.at[pl.ds(lo, NT0)])
    pltpu.sync_copy(zcnt_hbm.at[pl.ds(0, NT0)], cnt_sh.at[pl.ds(lo, NT0)])

    @pl.when(tid == NS - 1)
    def _():
        pltpu.sync_copy(zacc_hbm.at[pl.ds(0, NTT)],
                        acc_sh.at[pl.ds(NS * NT0, NTT)])
        pltpu.sync_copy(zcnt_hbm.at[pl.ds(0, NTT)],
                        cnt_sh.at[pl.ds(NS * NT0, NTT)])

    pltpu.sync_copy(ones_hbm, ones_v)
    pltpu.sync_copy(st_hbm, st_v)
    pltpu.sync_copy(dst3_hbm.at[wid], dstbuf)
    plsc.subcore_barrier()
    sc0 = st_v[0]
    sc1 = st_v[1]
    t0 = st_v[2]
    t1 = st_v[3]
    base = wid * EW

    def yslc(j):
        return pl.ds((j % NBY) * CH, CH)

    def yfire(j):
        pltpu.async_copy(y2_hbm.at[pl.ds(base + j * CH, CH)],
                         ybuf.at[yslc(j)], ysem.at[j % NBY])

    def ywait(j):
        pltpu.make_async_copy(y2_hbm.at[pl.ds(base + j * CH, CH)],
                              ybuf.at[yslc(j)], ysem.at[j % NBY]).wait()

    def mfire(j):
        sl = j % NBY
        pltpu.async_copy(ybuf.at[yslc(j)], acc_sh.at[dstbuf.at[j]],
                         msem.at[sl], add=True)
        pltpu.async_copy(ones_v, cnt_sh.at[dstbuf.at[j]],
                         msem.at[sl], add=True)

    def mwait(j):
        sl = j % NBY
        pltpu.make_async_copy(ybuf.at[yslc(j)], acc_sh.at[dstbuf.at[j]],
                              msem.at[sl]).wait()
        pltpu.make_async_copy(ones_v, cnt_sh.at[dstbuf.at[j]],
                              msem.at[sl]).wait()

    for j in range(KP6):
        yfire(j)

    def step(i, carry):
        j = i + KP6

        @pl.when(j < NCH)
        def _():
            @pl.when(j >= NBY)
            def _():
                mwait(j - NBY)

            yfire(j)

        ywait(i)
        yb = (i % NBY) * CH

        def row(q4, carry2):
            r0 = yb + q4 * 4
            for d in range(4):
                xa = ybuf[r0 + d, pl.ds(0, 16)] * sc0 + t0
                xb = ybuf[r0 + d, pl.ds(16, 16)] * sc1 + t1
                ybuf[r0 + d, pl.ds(0, 16)] = jnp.where(
                    xa > 0, xa, jnp.exp(xa) - 1.0)
                ybuf[r0 + d, pl.ds(16, 16)] = jnp.where(
                    xb > 0, xb, jnp.exp(xb) - 1.0)
            return carry2

        lax.fori_loop(0, CH // 4, row, 0)
        mfire(i)
        return carry

    lax.fori_loop(0, NCH, step, 0)
    for t in range(NBY):
        mwait(NCH - NBY + t)
    plsc.subcore_barrier()
    pltpu.sync_copy(acc_sh.at[pl.ds(lo, NT0)],
                    seg_out.at[c, pl.ds(lo, NT0)])
    pltpu.sync_copy(cnt_sh.at[pl.ds(lo, NT0)],
                    cnt_out.at[c, pl.ds(lo, NT0)])

    @pl.when(tid == NS - 1)
    def _():
        pltpu.sync_copy(acc_sh.at[pl.ds(NS * NT0, NTT)],
                        seg_out.at[c, pl.ds(NS * NT0, NTT)])
        pltpu.sync_copy(cnt_sh.at[pl.ds(NS * NT0, NTT)],
                        cnt_out.at[c, pl.ds(NS * NT0, NTT)])


def _k6(y2, dst3, st2):
    mesh = plsc.VectorSubcoreMesh(core_axis_name="c", subcore_axis_name="s")
    k = pl.kernel(
        _k6_body,
        out_type=[
            jax.ShapeDtypeStruct((NC, N, CQ), F32),
            jax.ShapeDtypeStruct((NC, N, 8), F32),
        ],
        mesh=mesh,
        scratch_types=[
            pltpu.VMEM_SHARED((N, CQ), F32),
            pltpu.VMEM_SHARED((N, 8), F32),
            pltpu.VMEM((NCH, CH), jnp.int32),
            pltpu.VMEM((NBY * CH, CQ), F32),
            pltpu.VMEM((CH, 8), F32),
            pltpu.VMEM((4, 16), F32),
            pltpu.SemaphoreType.DMA((NBY,)),
            pltpu.SemaphoreType.DMA((NBY,)),
        ],
    )
    st4 = st2.reshape(4, 16)
    ones = jnp.ones((CH, 8), F32)
    zacc = jnp.zeros((NT, CQ), F32)
    zcnt = jnp.zeros((NT, 8), F32)
    return k(y2, dst3, st4, ones, zacc, zcnt)


def _k7_body(nf_ref, hn_ref, seg_ref, cnt_ref,
             wna_ref, wnb_ref, bnm_ref, gnm_ref, bbnm_ref,
             wup_ref, bup_ref, g2_ref, bb2_ref,
             out_ref):
    seg = seg_ref[0] + seg_ref[1]
    cnt = cnt_ref[0] + cnt_ref[1]
    h_mean = seg / jnp.maximum(cnt[:, 0:1], 1.0)
    hn = hn_ref[...]
    y3 = jnp.dot(hn, wna_ref[...], preferred_element_type=F32) \
        + jnp.dot(h_mean, wnb_ref[...], preferred_element_type=F32) \
        + bnm_ref[...]
    m = jnp.mean(y3, axis=0, keepdims=True)
    v = jnp.mean(y3 * y3, axis=0, keepdims=True) - m * m
    sc = gnm_ref[...] * lax.rsqrt(v + EPS)
    t = bbnm_ref[...] - m * sc
    ho = _elu(y3 * sc + t)
    y4 = jnp.dot(ho, wup_ref[...], preferred_element_type=F32) + bup_ref[...]
    m2 = jnp.mean(y4, axis=0, keepdims=True)
    v2 = jnp.mean(y4 * y4, axis=0, keepdims=True) - m2 * m2
    sc2 = g2_ref[...] * lax.rsqrt(v2 + EPS)
    t2 = bb2_ref[...] - m2 * sc2
    out_ref[...] = _elu(y4 * sc2 + t2 + nf_ref[...])


def _k7(nf, hn, seg, cnt, wna, wnb, bnm, gnm, bbnm, wup, bup, g2, bb2):
    return pl.pallas_call(
        _k7_body,
        out_shape=jax.ShapeDtypeStruct((N, C), F32),
    )(nf, hn, seg, cnt, wna, wnb, bnm, gnm, bbnm, wup, bup, g2, bb2)


def _k8_body(ex_ref, ef_ref, gram_ref, sx_ref, w_ref, b_ref,
             g2_ref, bb2_ref, out_ref):
    w = w_ref[...]
    b = b_ref[...]
    sw = jnp.dot(sx_ref[...], w, preferred_element_type=F32)
    squ = jnp.sum(jnp.dot(gram_ref[...], w, preferred_element_type=F32) * w,
                  axis=0, keepdims=True) + 2.0 * b * sw + E * b * b
    mu = sw * (1.0 / E) + b
    vu = squ * (1.0 / E) - mu * mu
    scu = g2_ref[...] * lax.rsqrt(vu + EPS)
    tu = bb2_ref[...] + (b - mu) * scu
    u = jnp.dot(ex_ref[...].astype(F32), w, preferred_element_type=F32)
    out_ref[...] = _elu(u * scu + tu + ef_ref[...])


def _k8(ex, ef, gram, sx, w, b, g2, bb2):
    return pl.pallas_call(
        _k8_body,
        grid=(GE,),
        in_specs=[
            pl.BlockSpec((BE, CQ), lambda i: (i, 0)),
            pl.BlockSpec((BE, C), lambda i: (i, 0)),
            pl.BlockSpec((CQ, CQ), lambda i: (0, 0)),
            pl.BlockSpec((1, CQ), lambda i: (0, 0)),
            pl.BlockSpec((CQ, C), lambda i: (0, 0)),
            pl.BlockSpec((1, C), lambda i: (0, 0)),
            pl.BlockSpec((1, C), lambda i: (0, 0)),
            pl.BlockSpec((1, C), lambda i: (0, 0)),
        ],
        out_specs=pl.BlockSpec((BE, C), lambda i: (i, 0)),
        out_shape=jax.ShapeDtypeStruct((E, C), F32),
    )(ex, ef, gram, sx, w, b, g2, bb2)


def kernel(node_feats, edge_feats, edge_index, params):
    p = params
    src = edge_index[0]
    dst = edge_index[1]
    dst3 = dst.reshape(NW, NCH, CH)

    def r2(x):
        return x.reshape(1, -1)

    w1a, w1b, w1c = p["W_e1"][:CQ], p["W_e1"][CQ:2 * CQ], p["W_e1"][2 * CQ:]
    w2a, w2b = p["W_e2"][:CQ], p["W_e2"][CQ:]
    wna, wnb = p["W_nm"][:CQ], p["W_nm"][CQ:]

    z_e, st_e = _k1(edge_feats, p["W_down_e"], r2(p["b_down_e"]))
    hn, ac, b2 = _k2(node_feats, p["W_down_n"], r2(p["b_down_n"]),
                     r2(p["g1n"]), r2(p["bb1n"]),
                     w1a, w1b, w2a, r2(p["b_e1"]), r2(p["b_e2"]))
    gac, gb = _k3(ac, b2, src, dst)
    y1, st1 = _k4(z_e, gac, gb, st_e, r2(p["g1e"]), r2(p["bb1e"]), w1c)
    ex, y2, gram, sx, st2 = _k5(y1, gac, st1, r2(p["g_e1"]), r2(p["bb_e1"]),
                                w2b, r2(p["g_e2"]), r2(p["bb_e2"]))
    seg, cnt = _k6(y2, dst3, st2)
    node_out = _k7(node_feats, hn, seg, cnt, wna, wnb, r2(p["b_nm"]),
                   r2(p["g_nm"]), r2(p["bb_nm"]),
                   p["W_up_n"], r2(p["b_up_n"]), r2(p["g2n"]), r2(p["bb2n"]))
    edge_out = _k8(ex, edge_feats, gram, sx, p["W_up_e"], r2(p["b_up_e"]),
                   r2(p["g2e"]), r2(p["bb2e"]))
    return node_out, edge_out


# speedup vs baseline: 1.0019x; 1.0019x over previous
import functools

import jax
import jax.numpy as jnp
from jax import lax
from jax.experimental import pallas as pl
from jax.experimental.pallas import tpu as pltpu
from jax.experimental.pallas import tpu_sc as plsc

N = 10000
E = 320000
C = 128
CQ = 32
EPS = 1e-5

BE = 4000
GE = E // BE

NC = 2
NS = 16
NW = NC * NS
EW = E // NW
CH = 80
NCH = EW // CH
NT = N // NS
NT0 = 624
NTT = N - NS * NT0

F32 = jnp.float32
BF16 = jnp.bfloat16


def _elu(x):
    return jnp.where(x > 0, x, jnp.exp(x) - 1.0)


def _k1_body(ef_ref, w_ref, b_ref, z_ref, st_ref):
    i = pl.program_id(0)
    z = jnp.dot(ef_ref[...], w_ref[...], preferred_element_type=F32) + b_ref[...]
    z_ref[...] = z.astype(BF16)

    @pl.when(i == 0)
    def _():
        st_ref[...] = jnp.zeros_like(st_ref)

    s = jnp.sum(z, axis=0, keepdims=True)
    sq = jnp.sum(z * z, axis=0, keepdims=True)
    st_ref[...] += jnp.concatenate([s, sq], axis=0)


def _k1(ef, w, b):
    return pl.pallas_call(
        _k1_body,
        grid=(GE,),
        in_specs=[
            pl.BlockSpec((BE, C), lambda i: (i, 0)),
            pl.BlockSpec((C, CQ), lambda i: (0, 0)),
            pl.BlockSpec((1, CQ), lambda i: (0, 0)),
        ],
        out_specs=[
            pl.BlockSpec((BE, CQ), lambda i: (i, 0)),
            pl.BlockSpec((2, CQ), lambda i: (0, 0)),
        ],
        out_shape=[
            jax.ShapeDtypeStruct((E, CQ), BF16),
            jax.ShapeDtypeStruct((2, CQ), F32),
        ],
    )(ef, w, b)


def _k2_body(nf_ref, wdn_ref, bdn_ref, g_ref, bb_ref,
             w1a_ref, w1b_ref, w2a_ref, be1_ref, be2_ref,
             hn_ref, ac_ref, b2_ref):
    nf = nf_ref[...]
    z = jnp.dot(nf, wdn_ref[...], preferred_element_type=F32) + bdn_ref[...]
    m = jnp.mean(z, axis=0, keepdims=True)
    v = jnp.mean(z * z, axis=0, keepdims=True) - m * m
    sc = g_ref[...] * lax.rsqrt(v + EPS)
    t = bb_ref[...] - m * sc
    hn = _elu(z * sc + t)
    hn_ref[...] = hn
    a = jnp.dot(hn, w1a_ref[...], preferred_element_type=F32) + be1_ref[...]
    cc = jnp.dot(hn, w2a_ref[...], preferred_element_type=F32) + be2_ref[...]
    def pack2(x, y):
        xi = lax.bitcast_convert_type(x, jnp.int32) + 0x8000
        yi = lax.bitcast_convert_type(y, jnp.int32) + 0x8000
        hi = jnp.bitwise_and(xi, jnp.int32(-65536))
        lo = jnp.bitwise_and(jnp.right_shift(yi, 16), jnp.int32(0xFFFF))
        return jnp.bitwise_or(hi, lo)

    ac_ref[...] = pack2(a, cc)
    b2_ref[...] = jnp.dot(hn, w1b_ref[...], preferred_element_type=F32)


def _k2(nf, wdn, bdn, g, bb, w1a, w1b, w2a, be1, be2):
    return pl.pallas_call(
        _k2_body,
        out_shape=[
            jax.ShapeDtypeStruct((N, CQ), F32),
            jax.ShapeDtypeStruct((N, CQ), jnp.int32),
            jax.ShapeDtypeStruct((N, CQ), F32),
        ],
    )(nf, wdn, bdn, g, bb, w1a, w1b, w2a, be1, be2)


NSL = 3


def _k3_body(ac_hbm, b_hbm, src_hbm, dst_hbm,
             gac_out, gb_out,
             tabac_sh, tabb_sh,
             idxs, idxd, bufac, bufb, isem, gsem, osem):
    c = lax.axis_index("c")
    s = lax.axis_index("s")
    wid = s * NC + c
    base = wid * EW

    @pl.when(s == 0)
    def _():
        pltpu.sync_copy(ac_hbm, tabac_sh)

    @pl.when(s == 1)
    def _():
        pltpu.sync_copy(b_hbm, tabb_sh)

    plsc.subcore_barrier()

    def islc(j):
        return pl.ds((j % NSL) * CH, CH)

    def ifire(j):
        sl = j % NSL
        pltpu.async_copy(src_hbm.at[pl.ds(base + j * CH, CH)],
                         idxs.at[islc(j)], isem.at[sl])
        pltpu.async_copy(dst_hbm.at[pl.ds(base + j * CH, CH)],
                         idxd.at[islc(j)], isem.at[sl])

    def iwait(j):
        sl = j % NSL
        pltpu.make_async_copy(src_hbm.at[pl.ds(base + j * CH, CH)],
                              idxs.at[islc(j)], isem.at[sl]).wait()
        pltpu.make_async_copy(dst_hbm.at[pl.ds(base + j * CH, CH)],
                              idxd.at[islc(j)], isem.at[sl]).wait()

    def gfire(j):
        sl = j % NSL
        pltpu.async_copy(tabac_sh.at[idxs.at[islc(j)]], bufac.at[islc(j)],
                         gsem.at[sl])
        pltpu.async_copy(tabb_sh.at[idxd.at[islc(j)]], bufb.at[islc(j)],
                         gsem.at[sl])

    def gwait(j):
        sl = j % NSL
        pltpu.make_async_copy(tabac_sh.at[idxs.at[islc(j)]],
                              bufac.at[islc(j)], gsem.at[sl]).wait()
        pltpu.make_async_copy(tabb_sh.at[idxd.at[islc(j)]],
                              bufb.at[islc(j)], gsem.at[sl]).wait()

    def ofire(j):
        sl = j % NSL
        off = pl.ds(base + j * CH, CH)
        pltpu.async_copy(bufac.at[islc(j)], gac_out.at[off], osem.at[sl])
        pltpu.async_copy(bufb.at[islc(j)], gb_out.at[off], osem.at[sl])

    def owait(j):
        sl = j % NSL
        off = pl.ds(base + j * CH, CH)
        pltpu.make_async_copy(bufac.at[islc(j)], gac_out.at[off],
                              osem.at[sl]).wait()
        pltpu.make_async_copy(bufb.at[islc(j)], gb_out.at[off],
                              osem.at[sl]).wait()

    ifire(0)
    ifire(1)
    iwait(0)
    gfire(0)

    def step(i, carry):
        j2 = i + 2
        j1 = i + 1

        @pl.when(j2 < NCH)
        def _():
            ifire(j2)

        @pl.when(j1 < NCH)
        def _():
            iwait(j1)

            @pl.when(j1 >= NSL)
            def _():
                owait(j1 - NSL)

            gfire(j1)

        gwait(i)
        ofire(i)
        return carry

    lax.fori_loop(0, NCH, step, 0)
    for t in range(NSL):
        owait(NCH - NSL + t)


def _k3(ac, b, src, dst):
    mesh = plsc.VectorSubcoreMesh(core_axis_name="c", subcore_axis_name="s")
    k = pl.kernel(
        _k3_body,
        out_type=[
            jax.ShapeDtypeStruct((E, CQ), jnp.int32),
            jax.ShapeDtypeStruct((E, CQ), F32),
        ],
        mesh=mesh,
        scratch_types=[
            pltpu.VMEM_SHARED((N, CQ), jnp.int32),
            pltpu.VMEM_SHARED((N, CQ), F32),
            pltpu.VMEM((NSL * CH,), jnp.int32),
            pltpu.VMEM((NSL * CH,), jnp.int32),
            pltpu.VMEM((NSL * CH, CQ), jnp.int32),
            pltpu.VMEM((NSL * CH, CQ), F32),
            pltpu.SemaphoreType.DMA((NSL,)),
            pltpu.SemaphoreType.DMA((NSL,)),
            pltpu.SemaphoreType.DMA((NSL,)),
        ],
    )
    return k(ac, b, src, dst)


def _k4_body(z_ref, gac_ref, gb_ref, ste_ref, g_ref, bb_ref, w1c_ref,
             y1_ref, st_ref):
    i = pl.program_id(0)
    m = ste_ref[0:1, :] * (1.0 / E)
    v = ste_ref[1:2, :] * (1.0 / E) - m * m
    sc = g_ref[...] * lax.rsqrt(v + EPS)
    t = bb_ref[...] - m * sc
    he = _elu(z_ref[...].astype(F32) * sc + t)
    ga = lax.bitcast_convert_type(
        jnp.bitwise_and(gac_ref[...], jnp.int32(-65536)), F32)
    y1 = jnp.dot(he, w1c_ref[...], preferred_element_type=F32) \
        + ga + gb_ref[...]
    y1_ref[...] = y1.astype(BF16)

    @pl.when(i == 0)
    def _():
        st_ref[...] = jnp.zeros_like(st_ref)

    s = jnp.sum(y1, axis=0, keepdims=True)
    sq = jnp.sum(y1 * y1, axis=0, keepdims=True)
    st_ref[...] += jnp.concatenate([s, sq], axis=0)


def _k4(z, gac, gb, ste, g, bb, w1c):
    return pl.pallas_call(
        _k4_body,
        grid=(GE,),
        in_specs=[
            pl.BlockSpec((BE, CQ), lambda i: (i, 0)),
            pl.BlockSpec((BE, CQ), lambda i: (i, 0)),
            pl.BlockSpec((BE, CQ), lambda i: (i, 0)),
            pl.BlockSpec((2, CQ), lambda i: (0, 0)),
            pl.BlockSpec((1, CQ), lambda i: (0, 0)),
            pl.BlockSpec((1, CQ), lambda i: (0, 0)),
            pl.BlockSpec((CQ, CQ), lambda i: (0, 0)),
        ],
        out_specs=[
            pl.BlockSpec((BE, CQ), lambda i: (i, 0)),
            pl.BlockSpec((2, CQ), lambda i: (0, 0)),
        ],
        out_shape=[
            jax.ShapeDtypeStruct((E, CQ), BF16),
            jax.ShapeDtypeStruct((2, CQ), F32),
        ],
    )(z, gac, gb, ste, g, bb, w1c)


def _k5_body(y1_ref, gac_ref, st1_ref, g1_ref, bb1_ref, w2b_ref,
             g2_ref, bb2_ref,
             ex_ref, y2_ref, gram_ref, sx_ref, st2_ref, acc_ref):
    i = pl.program_id(0)
    m = st1_ref[0:1, :] * (1.0 / E)
    v = st1_ref[1:2, :] * (1.0 / E) - m * m
    sc = g1_ref[...] * lax.rsqrt(v + EPS)
    t = bb1_ref[...] - m * sc
    ex = _elu(y1_ref[...].astype(F32) * sc + t)
    ex_ref[...] = ex.astype(BF16)
    gc = lax.bitcast_convert_type(
        jnp.left_shift(gac_ref[...], 16), F32)
    y2 = jnp.dot(ex, w2b_ref[...], preferred_element_type=F32) + gc
    y2_ref[...] = y2

    @pl.when(i == 0)
    def _():
        gram_ref[...] = jnp.zeros_like(gram_ref)
        sx_ref[...] = jnp.zeros_like(sx_ref)
        acc_ref[...] = jnp.zeros_like(acc_ref)

    gram_ref[...] += lax.dot_general(
        ex, ex, (((0,), (0,)), ((), ())), preferred_element_type=F32)
    sx_ref[...] += jnp.sum(ex, axis=0, keepdims=True)
    s = jnp.sum(y2, axis=0, keepdims=True)
    sq = jnp.sum(y2 * y2, axis=0, keepdims=True)
    acc_ref[...] += jnp.concatenate([s, sq], axis=0)

    @pl.when(i == GE - 1)
    def _():
        m2 = acc_ref[0:1, :] * (1.0 / E)
        v2 = acc_ref[1:2, :] * (1.0 / E) - m2 * m2
        sc2 = g2_ref[...] * lax.rsqrt(v2 + EPS)
        t2 = bb2_ref[...] - m2 * sc2
        st2_ref[...] = jnp.concatenate([sc2, t2], axis=0)


def _k5(y1, gac, st1, g1, bb1, w2b, g2, bb2):
    return pl.pallas_call(
        _k5_body,
        grid=(GE,),
        in_specs=[
            pl.BlockSpec((BE, CQ), lambda i: (i, 0)),
            pl.BlockSpec((BE, CQ), lambda i: (i, 0)),
            pl.BlockSpec((2, CQ), lambda i: (0, 0)),
            pl.BlockSpec((1, CQ), lambda i: (0, 0)),
            pl.BlockSpec((1, CQ), lambda i: (0, 0)),
            pl.BlockSpec((CQ, CQ), lambda i: (0, 0)),
            pl.BlockSpec((1, CQ), lambda i: (0, 0)),
            pl.BlockSpec((1, CQ), lambda i: (0, 0)),
        ],
        out_specs=[
            pl.BlockSpec((BE, CQ), lambda i: (i, 0)),
            pl.BlockSpec((BE, CQ), lambda i: (i, 0)),
            pl.BlockSpec((CQ, CQ), lambda i: (0, 0)),
            pl.BlockSpec((1, CQ), lambda i: (0, 0)),
            pl.BlockSpec((2, CQ), lambda i: (0, 0)),
        ],
        out_shape=[
            jax.ShapeDtypeStruct((E, CQ), BF16),
            jax.ShapeDtypeStruct((E, CQ), F32),
            jax.ShapeDtypeStruct((CQ, CQ), F32),
            jax.ShapeDtypeStruct((1, CQ), F32),
            jax.ShapeDtypeStruct((2, CQ), F32),
        ],
        scratch_shapes=[pltpu.VMEM((2, CQ), F32)],
    )(y1, gac, st1, g1, bb1, w2b, g2, bb2)


NBY = 4
KP6 = 2


def _k6_body(y2_hbm, dst3_hbm, st_hbm, ones_hbm, zacc_hbm, zcnt_hbm,
             seg_out, cnt_out,
             acc_sh, cnt_sh, dstbuf, ybuf, ones_v, st_v, ysem, msem):
    c = lax.axis_index("c")
    s = lax.axis_index("s")
    wid = s * NC + c
    tid = s
    lo = tid * NT0
    pltpu.sync_copy(zacc_hbm.at[pl.ds(0, NT0)], acc_sh.at[pl.ds(lo, NT0)])
    pltpu.sync_copy(zcnt_hbm.at[pl.ds(0, NT0)], cnt_sh.at[pl.ds(lo, NT0)])

    @pl.when(tid == NS - 1)
    def _():
        pltpu.sync_copy(zacc_hbm.at[pl.ds(0, NTT)],
                        acc_sh.at[pl.ds(NS * NT0, NTT)])
        pltpu.sync_copy(zcnt_hbm.at[pl.ds(0, NTT)],
                        cnt_sh.at[pl.ds(NS * NT0, NTT)])

    pltpu.sync_copy(ones_hbm, ones_v)
    pltpu.sync_copy(st_hbm, st_v)
    pltpu.sync_copy(dst3_hbm.at[wid], dstbuf)
    plsc.subcore_barrier()
    sc0 = st_v[0]
    sc1 = st_v[1]
    t0 = st_v[2]
    t1 = st_v[3]
    base = wid * EW

    def yslc(j):
        return pl.ds((j % NBY) * CH, CH)

    def yfire(j):
        pltpu.async_copy(y2_hbm.at[pl.ds(base + j * CH, CH)],
                         ybuf.at[yslc(j)], ysem.at[j % NBY])

    def ywait(j):
        pltpu.make_async_copy(y2_hbm.at[pl.ds(base + j * CH, CH)],
                              ybuf.at[yslc(j)], ysem.at[j % NBY]).wait()

    def mfire(j):
        sl = j % NBY
        pltpu.async_copy(ybuf.at[yslc(j)], acc_sh.at[dstbuf.at[j]],
                         msem.at[sl], add=True)
        pltpu.async_copy(ones_v, cnt_sh.at[dstbuf.at[j]],
                         msem.at[sl], add=True)

    def mwait(j):
        sl = j % NBY
        pltpu.make_async_copy(ybuf.at[yslc(j)], acc_sh.at[dstbuf.at[j]],
                              msem.at[sl]).wait()
        pltpu.make_async_copy(ones_v, cnt_sh.at[dstbuf.at[j]],
                              msem.at[sl]).wait()

    for j in range(KP6):
        yfire(j)

    def step(i, carry):
        j = i + KP6

        @pl.when(j < NCH)
        def _():
            @pl.when(j >= NBY)
            def _():
                mwait(j - NBY)

            yfire(j)

        ywait(i)
        yb = (i % NBY) * CH

        def row(q8, carry2):
            r0 = yb + q8 * 8
            for d in range(8):
                xa = ybuf[r0 + d, pl.ds(0, 16)] * sc0 + t0
                xb = ybuf[r0 + d, pl.ds(16, 16)] * sc1 + t1
                ybuf[r0 + d, pl.ds(0, 16)] = jnp.where(
                    xa > 0, xa, jnp.exp(xa) - 1.0)
                ybuf[r0 + d, pl.ds(16, 16)] = jnp.where(
                    xb > 0, xb, jnp.exp(xb) - 1.0)
            return carry2

        lax.fori_loop(0, CH // 8, row, 0)
        mfire(i)
        return carry

    lax.fori_loop(0, NCH, step, 0)
    for t in range(NBY):
        mwait(NCH - NBY + t)
    plsc.subcore_barrier()
    pltpu.sync_copy(acc_sh.at[pl.ds(lo, NT0)],
                    seg_out.at[c, pl.ds(lo, NT0)])
    pltpu.sync_copy(cnt_sh.at[pl.ds(lo, NT0)],
                    cnt_out.at[c, pl.ds(lo, NT0)])

    @pl.when(tid == NS - 1)
    def _():
        pltpu.sync_copy(acc_sh.at[pl.ds(NS * NT0, NTT)],
                        seg_out.at[c, pl.ds(NS * NT0, NTT)])
        pltpu.sync_copy(cnt_sh.at[pl.ds(NS * NT0, NTT)],
                        cnt_out.at[c, pl.ds(NS * NT0, NTT)])


def _k6(y2, dst3, st2):
    mesh = plsc.VectorSubcoreMesh(core_axis_name="c", subcore_axis_name="s")
    k = pl.kernel(
        _k6_body,
        out_type=[
            jax.ShapeDtypeStruct((NC, N, CQ), F32),
            jax.ShapeDtypeStruct((NC, N, 8), F32),
        ],
        mesh=mesh,
        scratch_types=[
            pltpu.VMEM_SHARED((N, CQ), F32),
            pltpu.VMEM_SHARED((N, 8), F32),
            pltpu.VMEM((NCH, CH), jnp.int32),
            pltpu.VMEM((NBY * CH, CQ), F32),
            pltpu.VMEM((CH, 8), F32),
            pltpu.VMEM((4, 16), F32),
            pltpu.SemaphoreType.DMA((NBY,)),
            pltpu.SemaphoreType.DMA((NBY,)),
        ],
    )
    st4 = st2.reshape(4, 16)
    ones = jnp.ones((CH, 8), F32)
    zacc = jnp.zeros((NT, CQ), F32)
    zcnt = jnp.zeros((NT, 8), F32)
    return k(y2, dst3, st4, ones, zacc, zcnt)


def _k7_body(nf_ref, hn_ref, seg_ref, cnt_ref,
             wna_ref, wnb_ref, bnm_ref, gnm_ref, bbnm_ref,
             wup_ref, bup_ref, g2_ref, bb2_ref,
             out_ref):
    seg = seg_ref[0] + seg_ref[1]
    cnt = cnt_ref[0] + cnt_ref[1]
    h_mean = seg / jnp.maximum(cnt[:, 0:1], 1.0)
    hn = hn_ref[...]
    y3 = jnp.dot(hn, wna_ref[...], preferred_element_type=F32) \
        + jnp.dot(h_mean, wnb_ref[...], preferred_element_type=F32) \
        + bnm_ref[...]
    m = jnp.mean(y3, axis=0, keepdims=True)
    v = jnp.mean(y3 * y3, axis=0, keepdims=True) - m * m
    sc = gnm_ref[...] * lax.rsqrt(v + EPS)
    t = bbnm_ref[...] - m * sc
    ho = _elu(y3 * sc + t)
    y4 = jnp.dot(ho, wup_ref[...], preferred_element_type=F32) + bup_ref[...]
    m2 = jnp.mean(y4, axis=0, keepdims=True)
    v2 = jnp.mean(y4 * y4, axis=0, keepdims=True) - m2 * m2
    sc2 = g2_ref[...] * lax.rsqrt(v2 + EPS)
    t2 = bb2_ref[...] - m2 * sc2
    out_ref[...] = _elu(y4 * sc2 + t2 + nf_ref[...])


def _k7(nf, hn, seg, cnt, wna, wnb, bnm, gnm, bbnm, wup, bup, g2, bb2):
    return pl.pallas_call(
        _k7_body,
        out_shape=jax.ShapeDtypeStruct((N, C), F32),
    )(nf, hn, seg, cnt, wna, wnb, bnm, gnm, bbnm, wup, bup, g2, bb2)


def _k8_body(ex_ref, ef_ref, gram_ref, sx_ref, w_ref, b_ref,
             g2_ref, bb2_ref, out_ref):
    w = w_ref[...]
    b = b_ref[...]
    sw = jnp.dot(sx_ref[...], w, preferred_element_type=F32)
    squ = jnp.sum(jnp.dot(gram_ref[...], w, preferred_element_type=F32) * w,
                  axis=0, keepdims=True) + 2.0 * b * sw + E * b * b
    mu = sw * (1.0 / E) + b
    vu = squ * (1.0 / E) - mu * mu
    scu = g2_ref[...] * lax.rsqrt(vu + EPS)
    tu = bb2_ref[...] + (b - mu) * scu
    u = jnp.dot(ex_ref[...].astype(F32), w, preferred_element_type=F32)
    out_ref[...] = _elu(u * scu + tu + ef_ref[...])


def _k8(ex, ef, gram, sx, w, b, g2, bb2):
    return pl.pallas_call(
        _k8_body,
        grid=(GE,),
        in_specs=[
            pl.BlockSpec((BE, CQ), lambda i: (i, 0)),
            pl.BlockSpec((BE, C), lambda i: (i, 0)),
            pl.BlockSpec((CQ, CQ), lambda i: (0, 0)),
            pl.BlockSpec((1, CQ), lambda i: (0, 0)),
            pl.BlockSpec((CQ, C), lambda i: (0, 0)),
            pl.BlockSpec((1, C), lambda i: (0, 0)),
            pl.BlockSpec((1, C), lambda i: (0, 0)),
            pl.BlockSpec((1, C), lambda i: (0, 0)),
        ],
        out_specs=pl.BlockSpec((BE, C), lambda i: (i, 0)),
        out_shape=jax.ShapeDtypeStruct((E, C), F32),
    )(ex, ef, gram, sx, w, b, g2, bb2)


def kernel(node_feats, edge_feats, edge_index, params):
    p = params
    src = edge_index[0]
    dst = edge_index[1]
    dst3 = dst.reshape(NW, NCH, CH)

    def r2(x):
        return x.reshape(1, -1)

    w1a, w1b, w1c = p["W_e1"][:CQ], p["W_e1"][CQ:2 * CQ], p["W_e1"][2 * CQ:]
    w2a, w2b = p["W_e2"][:CQ], p["W_e2"][CQ:]
    wna, wnb = p["W_nm"][:CQ], p["W_nm"][CQ:]

    z_e, st_e = _k1(edge_feats, p["W_down_e"], r2(p["b_down_e"]))
    hn, ac, b2 = _k2(node_feats, p["W_down_n"], r2(p["b_down_n"]),
                     r2(p["g1n"]), r2(p["bb1n"]),
                     w1a, w1b, w2a, r2(p["b_e1"]), r2(p["b_e2"]))
    gac, gb = _k3(ac, b2, src, dst)
    y1, st1 = _k4(z_e, gac, gb, st_e, r2(p["g1e"]), r2(p["bb1e"]), w1c)
    ex, y2, gram, sx, st2 = _k5(y1, gac, st1, r2(p["g_e1"]), r2(p["bb_e1"]),
                                w2b, r2(p["g_e2"]), r2(p["bb_e2"]))
    seg, cnt = _k6(y2, dst3, st2)
    node_out = _k7(node_feats, hn, seg, cnt, wna, wnb, r2(p["b_nm"]),
                   r2(p["g_nm"]), r2(p["bb_nm"]),
                   p["W_up_n"], r2(p["b_up_n"]), r2(p["g2n"]), r2(p["bb2n"]))
    edge_out = _k8(ex, edge_feats, gram, sx, p["W_up_e"], r2(p["b_up_e"]),
                   r2(p["g2e"]), r2(p["bb2e"]))
    return node_out, edge_out
